# baseline (speedup 1.0000x reference)
import functools

import jax
import jax.numpy as jnp
from jax import lax
from jax.experimental import pallas as pl
from jax.experimental.pallas import tpu as pltpu
from jax.experimental.pallas import tpu_sc as plsc

_NC = 2
_NS = 16
_NW = _NC * _NS
_LANES = 16


def _make_pool(Bn, Ln, Dn):
    bpw = Bn // _NW
    nvec = Dn // _LANES
    mesh = plsc.VectorSubcoreMesh(
        core_axis_name="c", subcore_axis_name="s",
        num_cores=_NC, num_subcores=_NS)

    chunks = []
    off = 0
    while off < Ln:
        sz = min(128, Ln - off)
        chunks.append((off, sz))
        off += sz

    @functools.partial(
        pl.kernel,
        out_type=jax.ShapeDtypeStruct((Bn, Dn), jnp.float32),
        mesh=mesh,
        scratch_types=[
            pltpu.VMEM((Ln,), jnp.int32),
            pltpu.VMEM((Ln, Dn), jnp.float32),
            pltpu.VMEM((bpw, Dn), jnp.float32),
            pltpu.SemaphoreType.DMA,
        ],
        compiler_params=pltpu.CompilerParams(use_tc_tiling_on_sc=False),
    )
    def pool(data_hbm, table_hbm, out_hbm, idx_v, rows_v, out_v, sem):
        wid = lax.axis_index("s") * _NC + lax.axis_index("c")
        base = wid * bpw

        def body(g, carry):
            bidx = base + g
            pltpu.sync_copy(data_hbm.at[bidx], idx_v)
            cps = [
                pltpu.async_copy(
                    table_hbm.at[idx_v.at[pl.ds(o, sz)]],
                    rows_v.at[pl.ds(o, sz)], sem)
                for (o, sz) in chunks
            ]
            for cp in cps:
                cp.wait()

            def acc_body(l, acc):
                return tuple(
                    acc[k] + rows_v[l, pl.ds(k * _LANES, _LANES)]
                    for k in range(nvec))

            acc = lax.fori_loop(
                0, Ln, acc_body,
                tuple(jnp.zeros((_LANES,), jnp.float32) for _ in range(nvec)))
            for k in range(nvec):
                out_v[g, pl.ds(k * _LANES, _LANES)] = acc[k]
            return carry

        lax.fori_loop(0, bpw, body, 0)
        pltpu.sync_copy(out_v, out_hbm.at[pl.ds(base, bpw)])

    return pool


def _tc_finish(pooled, data, length, row0, Wp, bp):
    Bn, Dn = pooled.shape
    Ln = data.shape[1]
    OP = Wp.shape[0]
    BLK = 256

    def body(pooled_ref, data_ref, len_ref, row0_ref, w_ref, b_ref, out_ref):
        n0 = jnp.sum((data_ref[...] == 0).astype(jnp.float32), axis=1,
                     keepdims=True)
        acc = pooled_ref[...] - n0 * row0_ref[...]
        inv = 1.0 / len_ref[...].astype(jnp.float32)
        out = lax.dot_general(acc * inv, w_ref[...], (((1,), (1,)), ((), ())),
                              preferred_element_type=jnp.float32)
        out_ref[...] = out + b_ref[...]

    return pl.pallas_call(
        body,
        grid=(Bn // BLK,),
        in_specs=[
            pl.BlockSpec((BLK, Dn), lambda i: (i, 0)),
            pl.BlockSpec((BLK, Ln), lambda i: (i, 0)),
            pl.BlockSpec((BLK, 1), lambda i: (i, 0)),
            pl.BlockSpec((1, Dn), lambda i: (0, 0)),
            pl.BlockSpec((OP, Dn), lambda i: (0, 0)),
            pl.BlockSpec((1, OP), lambda i: (0, 0)),
        ],
        out_specs=pl.BlockSpec((BLK, OP), lambda i: (i, 0)),
        out_shape=jax.ShapeDtypeStruct((Bn, OP), jnp.float32),
    )(pooled, data, length.reshape(Bn, 1), row0, Wp, bp)


def kernel(data, length, embed_table, W, b):
    Bn, Ln = data.shape
    Dn = embed_table.shape[1]
    On = W.shape[0]
    data = data.astype(jnp.int32)

    pooled = _make_pool(Bn, Ln, Dn)(data, embed_table)

    OP = 8
    Wp = jnp.zeros((OP, Dn), jnp.float32).at[:On].set(W)
    bp = jnp.zeros((1, OP), jnp.float32).at[0, :On].set(b)
    row0 = embed_table[0:1]
    out = _tc_finish(pooled, data, length, row0, Wp, bp)
    return out[:, :On]
